# baseline (speedup 1.0000x reference)
import numpy as np
import jax
import jax.numpy as jnp
from jax.experimental import pallas as pl
from jax.experimental.pallas import tpu as pltpu

_MOD = 16


def _interp_mat(l_in, l_out):
    pos = np.arange(l_out, dtype=np.float64) * (l_in - 1) / (l_out - 1)
    lo = np.floor(pos).astype(np.int64)
    hi = np.minimum(lo + 1, l_in - 1)
    w = pos - lo
    m = np.zeros((l_out, l_in), np.float64)
    m[np.arange(l_out), lo] += 1.0 - w
    m[np.arange(l_out), hi] += w
    return m.astype(np.float32)


def _phase_mat():
    m = np.zeros((512, 512), np.float32)
    for p in range(4):
        for q in range(128):
            m[4 * q + p, 128 * p + q] = 1.0
    return m


def _detangle_mat(cin):
    m = np.zeros((3 * cin, 384), np.float32)
    for ci in range(cin):
        for k in range(3):
            m[ci * 3 + k, 128 * k + ci] = 1.0
    return m


_MI = _interp_mat(16, 32)
_PH = _phase_mat()
_P22 = _detangle_mat(22)
_P64 = _detangle_mat(64)
_P128 = _detangle_mat(128)


def _mm(a, b):
    return jax.lax.dot_general(a, b, (((1,), (0,)), ((), ())),
                               preferred_element_type=jnp.float32)


def _mm_tt(a, b):
    return jax.lax.dot_general(a, b, (((1,), (1,)), ((), ())),
                               preferred_element_type=jnp.float32)


def _cdot(x3, w):
    return jax.lax.dot_general(x3, w, (((1,), (1,)), ((), ())),
                               preferred_element_type=jnp.float32)


def _merge(t3):
    return t3.reshape(1024, t3.shape[2])


def _rd(x):
    r = pltpu.roll(x, 1, axis=0)
    idx = jax.lax.broadcasted_iota(jnp.int32, (x.shape[0], 1), 0)
    return jnp.where((idx % _MOD) == 0, 0.0, r)


def _ru(x):
    r = pltpu.roll(x, x.shape[0] - 1, axis=0)
    idx = jax.lax.broadcasted_iota(jnp.int32, (x.shape[0], 1), 0)
    return jnp.where((idx % _MOD) == (_MOD - 1), 0.0, r)


def _lrd(x):
    r = pltpu.roll(x, 1, axis=2)
    idx = jax.lax.broadcasted_iota(jnp.int32, (1, 1, x.shape[2]), 2)
    return jnp.where((idx % _MOD) == 0, 0.0, r)


def _lru(x):
    r = pltpu.roll(x, x.shape[2] - 1, axis=2)
    idx = jax.lax.broadcasted_iota(jnp.int32, (1, 1, x.shape[2]), 2)
    return jnp.where((idx % _MOD) == (_MOD - 1), 0.0, r)


def _encode_all(x_ref, w1f, wd1f, w2f, wd2f, wr1f, wr2f, wr3f, wff,
                caw1f, caw2f, wqf, wkf, wvf, wof,
                b1, g1v, bt1, m1v, v1v, br1v, bd1v, br2v,
                b2, g2v, bt2, m2v, v2v, bd2v, br3v, bfv,
                bqv, bkv, bvv, bov,
                phm, p22, p64, p128, mi, out_ref):
    ph = _mm(x_ref[...], phm[...])
    X0 = ph[:, 0:128].reshape(8, 22, 128)
    X1 = ph[:, 128:256].reshape(8, 22, 128)
    X2 = ph[:, 256:384].reshape(8, 22, 128)
    X3 = ph[:, 384:512].reshape(8, 22, 128)

    dw1 = _mm(w1f[...], p22[...])
    w10 = dw1[:, 0:22]
    w11 = dw1[:, 128:150]
    w12 = dw1[:, 256:278]
    dv = _mm(wd1f[...], p64[...])
    v0 = dv[:, 0:64]
    v1t = dv[:, 128:192]
    v2t = dv[:, 256:320]
    dc = _mm(w2f[...], p64[...])
    c0 = dc[:, 0:64]
    c1 = dc[:, 128:192]
    c2 = dc[:, 256:320]
    de = _mm(wd2f[...], p128[...])
    e0 = de[:, 0:128]
    e1 = de[:, 128:256]
    e2 = de[:, 256:384]

    s1 = g1v[...] * jax.lax.rsqrt(v1v[...] + 1e-5)
    b1e = (b1[...] - m1v[...]) * s1 + bt1[...]
    s2 = g2v[...] * jax.lax.rsqrt(v2v[...] + 1e-5)
    b2e = (b2[...] - m2v[...]) * s2 + bt2[...]

    wr1 = wr1f[...]
    br1 = br1v[...]

    h0 = _merge(_cdot(_lrd(X3), w10) + _cdot(X0, w11) + _cdot(X1, w12))
    h1 = _merge(_cdot(X0, w10) + _cdot(X1, w11) + _cdot(X2, w12))
    h2 = _merge(_cdot(X1, w10) + _cdot(X2, w11) + _cdot(X3, w12))
    h3 = _merge(_cdot(X2, w10) + _cdot(X3, w11) + _cdot(_lru(X0), w12))
    i0 = _merge(_cdot(X0, wr1)) + br1
    i1 = _merge(_cdot(X1, wr1)) + br1
    i2r = _merge(_cdot(X2, wr1)) + br1
    i3r = _merge(_cdot(X3, wr1)) + br1
    h0 = jnp.maximum(jnp.maximum(h0 * s1 + b1e, 0.0) + i0, 0.0)
    h1 = jnp.maximum(jnp.maximum(h1 * s1 + b1e, 0.0) + i1, 0.0)
    h2 = jnp.maximum(jnp.maximum(h2 * s1 + b1e, 0.0) + i2r, 0.0)
    h3 = jnp.maximum(jnp.maximum(h3 * s1 + b1e, 0.0) + i3r, 0.0)

    bd1r = bd1v[...]
    ye = _mm_tt(_rd(h3), v0) + _mm_tt(h0, v1t) + _mm_tt(h1, v2t) + bd1r
    yo = _mm_tt(h1, v0) + _mm_tt(h2, v1t) + _mm_tt(h3, v2t) + bd1r

    i2e = _mm_tt((i0 + i1) * 0.5, wr2f[...]) + br2v[...]
    i2o = _mm_tt((i2r + i3r) * 0.5, wr2f[...]) + br2v[...]

    he = _mm_tt(_rd(yo), c0) + _mm_tt(ye, c1) + _mm_tt(yo, c2)
    ho = _mm_tt(ye, c0) + _mm_tt(yo, c1) + _mm_tt(_ru(ye), c2)
    h4e = jnp.maximum(jnp.maximum(he * s2 + b2e, 0.0) + i2e, 0.0)
    h4o = jnp.maximum(jnp.maximum(ho * s2 + b2e, 0.0) + i2o, 0.0)

    z = _mm_tt(_rd(h4o), e0) + _mm_tt(h4e, e1) + _mm_tt(h4o, e2) + bd2v[...]
    i3 = _mm_tt((i2e + i2o) * 0.5, wr3f[...]) + br3v[...]
    z2 = jnp.maximum(z + i3, 0.0)

    z3 = z2.reshape(64, 16, 32)
    hi = jax.lax.dot_general(z3, mi[...], (((1,), (1,)), ((), ())),
                             preferred_element_type=jnp.float32)
    hf = jax.lax.dot_general(hi, wff[...], (((1,), (1,)), ((), ())),
                             preferred_element_type=jnp.float32)
    hf = hf + bfv[...][None, :, :]

    avg = jnp.mean(hf, axis=1)
    mx = jnp.max(hf, axis=1)
    ga = _mm_tt(jnp.maximum(_mm_tt(avg, caw1f[...]), 0.0), caw2f[...])
    gm = _mm_tt(jnp.maximum(_mm_tt(mx, caw1f[...]), 0.0), caw2f[...])
    gate = jax.nn.sigmoid(ga + gm)
    o = jnp.sum(hf * gate[:, None, :], axis=2)

    o3 = o.reshape(8, 8, 32)
    q = jax.lax.dot_general(o3, wqf[...], (((2,), (1,)), ((), ())),
                            preferred_element_type=jnp.float32) \
        + bqv[...][None, :, :]
    k = jax.lax.dot_general(o3, wkf[...], (((2,), (1,)), ((), ())),
                            preferred_element_type=jnp.float32) \
        + bkv[...][None, :, :]
    v = jax.lax.dot_general(o3, wvf[...], (((2,), (1,)), ((), ())),
                            preferred_element_type=jnp.float32) \
        + bvv[...][None, :, :]
    sc = jax.lax.dot_general(q, k, (((2,), (2,)), ((0,), (0,))),
                             preferred_element_type=jnp.float32) * 0.125
    sc = sc - jnp.max(sc, axis=2, keepdims=True)
    es = jnp.exp(sc)
    p = es / jnp.sum(es, axis=2, keepdims=True)
    wvv = jax.lax.dot_general(p, v, (((2,), (1,)), ((0,), (0,))),
                              preferred_element_type=jnp.float32)
    pooled = jnp.mean(wvv, axis=1)
    out_ref[...] = _mm_tt(pooled, wof[...]) + bov[...]


def kernel(x, params):
    p = params
    f32 = jnp.float32

    def row(b):
        return b.reshape(1, b.shape[0])

    ops = [
        p['w1'].reshape(64, 66), p['wd1'].reshape(64, 192),
        p['w2'].reshape(128, 192), p['wd2'].reshape(32, 384),
        p['wr1'].reshape(64, 22), p['wr2'].reshape(128, 64),
        p['wr3'].reshape(32, 128), p['wf'].reshape(22, 32),
        p['ca_w1'], p['ca_w2'], p['wq'], p['wk'], p['wv'], p['wo'],
        row(p['b1']), row(p['bn1_g']), row(p['bn1_b']), row(p['bn1_m']),
        row(p['bn1_v']), row(p['br1']), row(p['bd1']), row(p['br2']),
        row(p['b2']), row(p['bn2_g']), row(p['bn2_b']), row(p['bn2_m']),
        row(p['bn2_v']), row(p['bd2']), row(p['br3']), row(p['bf']),
        row(p['bq']), row(p['bk']), row(p['bv']), row(p['bo']),
        jnp.asarray(_PH), jnp.asarray(_P22), jnp.asarray(_P64),
        jnp.asarray(_P128), jnp.asarray(_MI),
    ]

    xflat = x.reshape(176, 512)

    const_specs = [
        pl.BlockSpec(o.shape, lambda _n=o.ndim: (0,) * _n) for o in ops
    ]
    out = pl.pallas_call(
        _encode_all,
        in_specs=[pl.BlockSpec((176, 512), lambda: (0, 0))] + const_specs,
        out_specs=pl.BlockSpec((8, 64), lambda: (0, 0)),
        out_shape=jax.ShapeDtypeStruct((8, 64), f32),
    )(xflat, *ops)
    return out


# speedup vs baseline: 1.3135x; 1.3135x over previous
import numpy as np
import jax
import jax.numpy as jnp
from jax.experimental import pallas as pl
from jax.experimental.pallas import tpu as pltpu

_MOD = 16


def _interp_mat(l_in, l_out):
    pos = np.arange(l_out, dtype=np.float64) * (l_in - 1) / (l_out - 1)
    lo = np.floor(pos).astype(np.int64)
    hi = np.minimum(lo + 1, l_in - 1)
    w = pos - lo
    m = np.zeros((l_out, l_in), np.float64)
    m[np.arange(l_out), lo] += 1.0 - w
    m[np.arange(l_out), hi] += w
    return m.astype(np.float32)


_MI = _interp_mat(16, 32)

_BIAS_ORDER = ['b1', 'bn1_g', 'bn1_b', 'bn1_m', 'bn1_v', 'br1', 'bd1', 'br2',
               'b2', 'bn2_g', 'bn2_b', 'bn2_m', 'bn2_v', 'bd2', 'br3', 'bf',
               'bq', 'bk', 'bv', 'bo']
_BROW = {nm: i for i, nm in enumerate(_BIAS_ORDER)}


def _mm_tt(a, b):
    return jax.lax.dot_general(a, b, (((1,), (1,)), ((), ())),
                               preferred_element_type=jnp.float32)


def _cdot(x3, w):
    return jax.lax.dot_general(x3, w, (((1,), (1,)), ((), ())),
                               preferred_element_type=jnp.float32)


def _merge(t3):
    return t3.reshape(1024, t3.shape[2])


def _rd(x):
    r = pltpu.roll(x, 1, axis=0)
    idx = jax.lax.broadcasted_iota(jnp.int32, (x.shape[0], 1), 0)
    return jnp.where((idx % _MOD) == 0, 0.0, r)


def _ru(x):
    r = pltpu.roll(x, x.shape[0] - 1, axis=0)
    idx = jax.lax.broadcasted_iota(jnp.int32, (x.shape[0], 1), 0)
    return jnp.where((idx % _MOD) == (_MOD - 1), 0.0, r)


def _lrd(x):
    r = pltpu.roll(x, 1, axis=2)
    idx = jax.lax.broadcasted_iota(jnp.int32, (1, 1, x.shape[2]), 2)
    return jnp.where((idx % _MOD) == 0, 0.0, r)


def _lru(x):
    r = pltpu.roll(x, x.shape[2] - 1, axis=2)
    idx = jax.lax.broadcasted_iota(jnp.int32, (1, 1, x.shape[2]), 2)
    return jnp.where((idx % _MOD) == (_MOD - 1), 0.0, r)


def _encode_all(x_ref, pk_ref, out_ref):
    X0 = x_ref[0]
    X1 = x_ref[1]
    X2 = x_ref[2]
    X3 = x_ref[3]

    pk = pk_ref[...]
    w10 = pk[0, 0:64, 0:22]
    w11 = pk[1, 0:64, 0:22]
    w12 = pk[2, 0:64, 0:22]
    v0 = pk[3, 0:64, 0:64]
    v1t = pk[4, 0:64, 0:64]
    v2t = pk[5, 0:64, 0:64]
    c0 = pk[6, 0:128, 0:64]
    c1 = pk[7, 0:128, 0:64]
    c2 = pk[8, 0:128, 0:64]
    e0 = pk[9, 0:32, 0:128]
    e1 = pk[10, 0:32, 0:128]
    e2 = pk[11, 0:32, 0:128]
    wr1 = pk[12, 0:64, 0:22]
    wr2 = pk[13, 0:128, 0:64]
    wr3 = pk[14, 0:32, 0:128]
    wf = pk[15, 0:22, 0:32]
    caw1 = pk[16, 0:11, 0:22]
    caw2 = pk[17, 0:22, 0:11]
    wq = pk[18, 0:64, 0:32]
    wk = pk[19, 0:64, 0:32]
    wv = pk[20, 0:64, 0:32]
    wo = pk[21, 0:64, 0:64]
    mi = pk[22, 0:32, 0:16]
    bt = pk[23]

    def brow(nm, c):
        o = _BROW[nm]
        return bt[o:o + 1, 0:c]

    s1 = brow('bn1_g', 64) * jax.lax.rsqrt(brow('bn1_v', 64) + 1e-5)
    b1e = (brow('b1', 64) - brow('bn1_m', 64)) * s1 + brow('bn1_b', 64)
    s2 = brow('bn2_g', 128) * jax.lax.rsqrt(brow('bn2_v', 128) + 1e-5)
    b2e = (brow('b2', 128) - brow('bn2_m', 128)) * s2 + brow('bn2_b', 128)
    br1 = brow('br1', 64)

    h0 = _merge(_cdot(_lrd(X3), w10) + _cdot(X0, w11) + _cdot(X1, w12))
    h1 = _merge(_cdot(X0, w10) + _cdot(X1, w11) + _cdot(X2, w12))
    h2 = _merge(_cdot(X1, w10) + _cdot(X2, w11) + _cdot(X3, w12))
    h3 = _merge(_cdot(X2, w10) + _cdot(X3, w11) + _cdot(_lru(X0), w12))
    i0 = _merge(_cdot(X0, wr1)) + br1
    i1 = _merge(_cdot(X1, wr1)) + br1
    i2r = _merge(_cdot(X2, wr1)) + br1
    i3r = _merge(_cdot(X3, wr1)) + br1
    h0 = jnp.maximum(jnp.maximum(h0 * s1 + b1e, 0.0) + i0, 0.0)
    h1 = jnp.maximum(jnp.maximum(h1 * s1 + b1e, 0.0) + i1, 0.0)
    h2 = jnp.maximum(jnp.maximum(h2 * s1 + b1e, 0.0) + i2r, 0.0)
    h3 = jnp.maximum(jnp.maximum(h3 * s1 + b1e, 0.0) + i3r, 0.0)

    bd1r = brow('bd1', 64)
    ye = _mm_tt(_rd(h3), v0) + _mm_tt(h0, v1t) + _mm_tt(h1, v2t) + bd1r
    yo = _mm_tt(h1, v0) + _mm_tt(h2, v1t) + _mm_tt(h3, v2t) + bd1r

    br2 = brow('br2', 128)
    i2e = _mm_tt((i0 + i1) * 0.5, wr2) + br2
    i2o = _mm_tt((i2r + i3r) * 0.5, wr2) + br2

    he = _mm_tt(_rd(yo), c0) + _mm_tt(ye, c1) + _mm_tt(yo, c2)
    ho = _mm_tt(ye, c0) + _mm_tt(yo, c1) + _mm_tt(_ru(ye), c2)
    h4e = jnp.maximum(jnp.maximum(he * s2 + b2e, 0.0) + i2e, 0.0)
    h4o = jnp.maximum(jnp.maximum(ho * s2 + b2e, 0.0) + i2o, 0.0)

    z = _mm_tt(_rd(h4o), e0) + _mm_tt(h4e, e1) + _mm_tt(h4o, e2) \
        + brow('bd2', 32)
    i3 = _mm_tt((i2e + i2o) * 0.5, wr3) + brow('br3', 32)
    z2 = jnp.maximum(z + i3, 0.0)

    z3 = z2.reshape(64, 16, 32)
    hi = jax.lax.dot_general(z3, mi, (((1,), (1,)), ((), ())),
                             preferred_element_type=jnp.float32)
    hf = jax.lax.dot_general(hi, wf, (((1,), (1,)), ((), ())),
                             preferred_element_type=jnp.float32)
    hf = hf + brow('bf', 22)[None, :, :]

    avg = jnp.mean(hf, axis=1)
    mx = jnp.max(hf, axis=1)
    ga = _mm_tt(jnp.maximum(_mm_tt(avg, caw1), 0.0), caw2)
    gm = _mm_tt(jnp.maximum(_mm_tt(mx, caw1), 0.0), caw2)
    gate = jax.nn.sigmoid(ga + gm)
    o = jnp.sum(hf * gate[:, None, :], axis=2)

    o3 = o.reshape(8, 8, 32)
    q = jax.lax.dot_general(o3, wq, (((2,), (1,)), ((), ())),
                            preferred_element_type=jnp.float32) \
        + brow('bq', 64)[None, :, :]
    k = jax.lax.dot_general(o3, wk, (((2,), (1,)), ((), ())),
                            preferred_element_type=jnp.float32) \
        + brow('bk', 64)[None, :, :]
    v = jax.lax.dot_general(o3, wv, (((2,), (1,)), ((), ())),
                            preferred_element_type=jnp.float32) \
        + brow('bv', 64)[None, :, :]
    sc = jax.lax.dot_general(q, k, (((2,), (2,)), ((0,), (0,))),
                             preferred_element_type=jnp.float32) * 0.125
    sc = sc - jnp.max(sc, axis=2, keepdims=True)
    es = jnp.exp(sc)
    p = es / jnp.sum(es, axis=2, keepdims=True)
    wvv = jax.lax.dot_general(p, v, (((2,), (1,)), ((0,), (0,))),
                              preferred_element_type=jnp.float32)
    pooled = jnp.mean(wvv, axis=1)
    out_ref[...] = _mm_tt(pooled, wo) + brow('bo', 64)


def kernel(x, params):
    p = params
    f32 = jnp.float32

    def tile(t):
        return jnp.pad(t, ((0, 128 - t.shape[0]), (0, 128 - t.shape[1])))

    btile = jnp.stack([jnp.pad(p[nm], (0, 128 - p[nm].shape[0]))
                       for nm in _BIAS_ORDER]
                      + [jnp.zeros(128, f32)] * (128 - len(_BIAS_ORDER)))

    pk = jnp.stack([
        tile(p['w1'][:, :, 0]), tile(p['w1'][:, :, 1]), tile(p['w1'][:, :, 2]),
        tile(p['wd1'][:, :, 0]), tile(p['wd1'][:, :, 1]), tile(p['wd1'][:, :, 2]),
        tile(p['w2'][:, :, 0]), tile(p['w2'][:, :, 1]), tile(p['w2'][:, :, 2]),
        tile(p['wd2'][:, :, 0]), tile(p['wd2'][:, :, 1]), tile(p['wd2'][:, :, 2]),
        tile(p['wr1'][:, :, 0]), tile(p['wr2'][:, :, 0]), tile(p['wr3'][:, :, 0]),
        tile(p['wf'][:, :, 0]), tile(p['ca_w1']), tile(p['ca_w2']),
        tile(p['wq']), tile(p['wk']), tile(p['wv']), tile(p['wo']),
        tile(jnp.asarray(_MI)), btile,
    ])

    xt = jnp.transpose(x.reshape(8, 22, 128, 4), (3, 0, 1, 2))

    out = pl.pallas_call(
        _encode_all,
        in_specs=[pl.BlockSpec((4, 8, 22, 128), lambda: (0, 0, 0, 0)),
                  pl.BlockSpec((24, 128, 128), lambda: (0, 0, 0))],
        out_specs=pl.BlockSpec((8, 64), lambda: (0, 0)),
        out_shape=jax.ShapeDtypeStruct((8, 64), f32),
    )(xt, pk)
    return out
